# baseline (speedup 1.0000x reference)
import jax
import jax.numpy as jnp
from jax import lax
from jax.experimental import pallas as pl
from jax.experimental.pallas import tpu as pltpu
from jax.experimental.pallas import tpu_sc as plsc

N_USERS = 5000
N_PRODS = 5000
N_NODES = N_USERS + N_PRODS
NACC = 10240
EDGES = 320000
NSUB = 16
CHUNK = 128
CHUNKS_PER_SUB = 160
SUPER = 40
EPAD = NSUB * CHUNKS_PER_SUB * CHUNK
D = 128
DH = 64
HROWS = NACC // 128
ROWS_PER_SUB = NACC // NSUB
STAGE_ROWS = N_NODES // NSUB
NTBL = N_NODES + 48

_f32 = jnp.float32
_i32 = jnp.int32


_sc_mesh = plsc.VectorSubcoreMesh(core_axis_name="c", subcore_axis_name="s")


def _sc_body(y_hbm, src_hbm, dst_hbm, zeros_hbm, idx80_hbm,
             out_feat, out_deg,
             src_v, dst_v, r0, r1, hist_v, idx80_v,
             y_spm, accf, accd, g0, g1, s0, s1):
    c = lax.axis_index("c")
    s = lax.axis_index("s")
    base = s * ROWS_PER_SUB

    pltpu.sync_copy(y_hbm.at[pl.ds(s * STAGE_ROWS, STAGE_ROWS),
                             pl.ds(c * DH, DH)],
                    y_spm.at[pl.ds(s * STAGE_ROWS, STAGE_ROWS)])
    pltpu.sync_copy(idx80_hbm, idx80_v)

    @pl.when(s == 0)
    def _zero_accd():
        pltpu.sync_copy(zeros_hbm, accd)

    z16 = jnp.zeros((16,), _f32)

    def _zrow(i, carry):
        for j in range(DH // 16):
            r0[i, j * 16:(j + 1) * 16] = z16
        return carry

    lax.fori_loop(0, CHUNK, _zrow, 0)

    def _zhist(i, carry):
        for j in range(128 // 16):
            hist_v[i, j * 16:(j + 1) * 16] = z16
        return carry

    lax.fori_loop(0, HROWS, _zhist, 0)

    for k in range(ROWS_PER_SUB // CHUNK):
        pltpu.sync_copy(r0, accf.at[pl.ds(base + k * CHUNK, CHUNK)])

    plsc.subcore_barrier()

    ones16 = jnp.full((16,), 1.0, _f32)

    def _hist_update(a):
        for j in range(CHUNK // 16):
            dv = dst_v[a, j * 16:(j + 1) * 16]
            plsc.addupdate_scatter(
                hist_v, [lax.shift_right_logical(dv, 7),
                         lax.bitwise_and(dv, 127)], ones16)

    def _super(t, carry):
        pltpu.sync_copy(src_hbm.at[s, pl.ds(t * SUPER, SUPER)], src_v)
        pltpu.sync_copy(dst_hbm.at[s, pl.ds(t * SUPER, SUPER)], dst_v)

        def _group(g, carry2):
            a = g * 2
            ga = pltpu.async_copy(y_spm.at[src_v.at[a]], r0, g0)
            gb = pltpu.async_copy(y_spm.at[src_v.at[a + 1]], r1, g1)
            _hist_update(a)
            _hist_update(a + 1)
            ga.wait()
            sa = pltpu.async_copy(r0, accf.at[dst_v.at[a]], s0, add=True)
            gb.wait()
            sb = pltpu.async_copy(r1, accf.at[dst_v.at[a + 1]], s1, add=True)
            sa.wait()
            sb.wait()
            return carry2

        lax.fori_loop(0, SUPER // 2, _group, 0)
        return carry

    lax.fori_loop(0, CHUNKS_PER_SUB // SUPER, _super, 0)

    pltpu.sync_copy(hist_v, accd.at[idx80_v], add=True)

    plsc.subcore_barrier()

    pltpu.sync_copy(accf.at[pl.ds(base, ROWS_PER_SUB)],
                    out_feat.at[pl.ds(base, ROWS_PER_SUB), pl.ds(c * DH, DH)])

    @pl.when(s == 0)
    def _write_deg():
        pltpu.sync_copy(accd, out_deg.at[c])


_sc_segsum = pl.kernel(
    _sc_body,
    out_type=(
        jax.ShapeDtypeStruct((NACC, D), _f32),
        jax.ShapeDtypeStruct((2, HROWS, 128), _f32),
    ),
    mesh=_sc_mesh,
    scratch_types=[
        pltpu.VMEM((SUPER, CHUNK), _i32),
        pltpu.VMEM((SUPER, CHUNK), _i32),
        pltpu.VMEM((CHUNK, DH), _f32),
        pltpu.VMEM((CHUNK, DH), _f32),
        pltpu.VMEM((HROWS, 128), _f32),
        pltpu.VMEM((HROWS,), _i32),
        pltpu.VMEM_SHARED((NTBL, DH), _f32),
        pltpu.VMEM_SHARED((NACC, DH), _f32),
        pltpu.VMEM_SHARED((HROWS, 128), _f32),
        pltpu.SemaphoreType.DMA,
        pltpu.SemaphoreType.DMA,
        pltpu.SemaphoreType.DMA,
        pltpu.SemaphoreType.DMA,
    ],
    compiler_params=pltpu.CompilerParams(use_tc_tiling_on_sc=False,
                                         needs_layout_passes=False),
)


_RB = 1000
_HI = jax.lax.Precision.DEFAULT


def _tc_rmm_body(x_ref, w_ref, o_ref):
    o_ref[...] = jnp.dot(x_ref[...], w_ref[...], preferred_element_type=_f32,
                         precision=_HI)


def _tc_rmm(x, wT):
    return pl.pallas_call(
        _tc_rmm_body,
        grid=(N_NODES // _RB,),
        in_specs=[
            pl.BlockSpec((_RB, D), lambda i: (i, 0)),
            pl.BlockSpec((D, D), lambda i: (0, 0)),
        ],
        out_specs=pl.BlockSpec((_RB, D), lambda i: (i, 0)),
        out_shape=jax.ShapeDtypeStruct((N_NODES, D), _f32),
    )(x, wT)


def _tc_mid_body(pf_ref, dv_ref, b_ref, r_ref, wlT_ref,
                 h_ref, y2_ref):
    inv = 1.0 / jnp.maximum(dv_ref[...], 1.0)
    h = jnp.maximum(pf_ref[...] * inv + b_ref[...] + r_ref[...], 0.0)
    h_ref[...] = h
    y2_ref[...] = jnp.dot(h, wlT_ref[...], preferred_element_type=_f32,
                          precision=_HI)


def _tc_mid(pf, dv, b, r, wlT):
    return pl.pallas_call(
        _tc_mid_body,
        grid=(N_NODES // _RB,),
        in_specs=[
            pl.BlockSpec((_RB, D), lambda i: (i, 0)),
            pl.BlockSpec((_RB, 1), lambda i: (i, 0)),
            pl.BlockSpec((1, D), lambda i: (0, 0)),
            pl.BlockSpec((_RB, D), lambda i: (i, 0)),
            pl.BlockSpec((D, D), lambda i: (0, 0)),
        ],
        out_specs=[
            pl.BlockSpec((_RB, D), lambda i: (i, 0)),
            pl.BlockSpec((_RB, D), lambda i: (i, 0)),
        ],
        out_shape=[
            jax.ShapeDtypeStruct((N_NODES, D), _f32),
            jax.ShapeDtypeStruct((N_NODES, D), _f32),
        ],
    )(pf, dv, b, r, wlT)


def _tc_post_body(pf_ref, dv_ref, b_ref, r_ref, out_ref):
    inv = 1.0 / jnp.maximum(dv_ref[...], 1.0)
    out_ref[0] = pf_ref[...] * inv + b_ref[...] + r_ref[...]


def _tc_post(pf, dv, b, r):
    return pl.pallas_call(
        _tc_post_body,
        grid=(N_NODES // _RB,),
        in_specs=[
            pl.BlockSpec((_RB, D), lambda i: (i, 0)),
            pl.BlockSpec((_RB, 1), lambda i: (i, 0)),
            pl.BlockSpec((1, D), lambda i: (0, 0)),
            pl.BlockSpec((_RB, D), lambda i: (i, 0)),
        ],
        out_specs=pl.BlockSpec((1, _RB, D), lambda i: (i // 5, i % 5, 0)),
        out_shape=jax.ShapeDtypeStruct((2, N_USERS, D), _f32),
    )(pf, dv, b, r)


def kernel(edge_index, num_users, user_emb, prod_emb, W1l, b1l, W1r, W2l,
           b2l, W2r):
    x = jnp.concatenate([user_emb, prod_emb], axis=0)

    ep = jnp.pad(edge_index, ((0, 0), (0, EPAD - EDGES)),
                 constant_values=N_NODES)
    srcp = ep[0].reshape(NSUB, CHUNKS_PER_SUB, CHUNK)
    dstp = ep[1].reshape(NSUB, CHUNKS_PER_SUB, CHUNK)

    zeros80 = jnp.zeros((HROWS, 128), _f32)
    idx80 = jnp.arange(HROWS, dtype=_i32)

    b1 = b1l.reshape(1, D)
    b2 = b2l.reshape(1, D)

    y1 = _tc_rmm(x, W1l.T)
    pf1, pd1 = _sc_segsum(y1, srcp, dstp, zeros80, idx80)
    r1 = _tc_rmm(x, W1r.T)
    degv = pd1[0].reshape(NACC, 1)[:N_NODES]
    h, y2 = _tc_mid(pf1, degv, b1, r1, W2l.T)

    pf2, _ = _sc_segsum(y2, srcp, dstp, zeros80, idx80)
    r2 = _tc_rmm(h, W2r.T)
    out = _tc_post(pf2, degv, b2, r2)

    return (out[0], out[1])


# speedup vs baseline: 1.0087x; 1.0087x over previous
import jax
import jax.numpy as jnp
from jax import lax
from jax.experimental import pallas as pl
from jax.experimental.pallas import tpu as pltpu
from jax.experimental.pallas import tpu_sc as plsc

N_USERS = 5000
N_PRODS = 5000
N_NODES = N_USERS + N_PRODS
NACC = 10240
EDGES = 320000
NSUB = 16
CHUNK = 128
CHUNKS_PER_SUB = 160
SUPER = 80
EPAD = NSUB * CHUNKS_PER_SUB * CHUNK
D = 128
DH = 64
HROWS = NACC // 128
ROWS_PER_SUB = NACC // NSUB
STAGE_ROWS = N_NODES // NSUB
NTBL = N_NODES + 48

_f32 = jnp.float32
_i32 = jnp.int32


_sc_mesh = plsc.VectorSubcoreMesh(core_axis_name="c", subcore_axis_name="s")


def _sc_body(y_hbm, src_hbm, dst_hbm, zeros_hbm, idx80_hbm,
             out_feat, out_deg,
             src_v, dst_v, r0, r1, hist_v, idx80_v,
             y_spm, accf, accd, g0, g1, s0, s1):
    c = lax.axis_index("c")
    s = lax.axis_index("s")
    base = s * ROWS_PER_SUB

    pltpu.sync_copy(y_hbm.at[pl.ds(s * STAGE_ROWS, STAGE_ROWS),
                             pl.ds(c * DH, DH)],
                    y_spm.at[pl.ds(s * STAGE_ROWS, STAGE_ROWS)])
    pltpu.sync_copy(idx80_hbm, idx80_v)

    @pl.when(s == 0)
    def _zero_accd():
        pltpu.sync_copy(zeros_hbm, accd)

    z16 = jnp.zeros((16,), _f32)

    def _zrow(i, carry):
        for j in range(DH // 16):
            r0[i, j * 16:(j + 1) * 16] = z16
        return carry

    lax.fori_loop(0, CHUNK, _zrow, 0)

    def _zhist(i, carry):
        for j in range(128 // 16):
            hist_v[i, j * 16:(j + 1) * 16] = z16
        return carry

    lax.fori_loop(0, HROWS, _zhist, 0)

    for k in range(ROWS_PER_SUB // CHUNK):
        pltpu.sync_copy(r0, accf.at[pl.ds(base + k * CHUNK, CHUNK)])

    plsc.subcore_barrier()

    ones16 = jnp.full((16,), 1.0, _f32)

    def _hist_update(a):
        for j in range(CHUNK // 16):
            dv = dst_v[a, j * 16:(j + 1) * 16]
            plsc.addupdate_scatter(
                hist_v, [lax.shift_right_logical(dv, 7),
                         lax.bitwise_and(dv, 127)], ones16)

    def _super(t, carry):
        pltpu.sync_copy(src_hbm.at[s, pl.ds(t * SUPER, SUPER)], src_v)
        pltpu.sync_copy(dst_hbm.at[s, pl.ds(t * SUPER, SUPER)], dst_v)

        def _group(g, carry2):
            a = g * 2
            ga = pltpu.async_copy(y_spm.at[src_v.at[a]], r0, g0)
            gb = pltpu.async_copy(y_spm.at[src_v.at[a + 1]], r1, g1)
            _hist_update(a)
            _hist_update(a + 1)
            ga.wait()
            sa = pltpu.async_copy(r0, accf.at[dst_v.at[a]], s0, add=True)
            gb.wait()
            sb = pltpu.async_copy(r1, accf.at[dst_v.at[a + 1]], s1, add=True)
            sa.wait()
            sb.wait()
            return carry2

        lax.fori_loop(0, SUPER // 2, _group, 0)
        return carry

    lax.fori_loop(0, CHUNKS_PER_SUB // SUPER, _super, 0)

    pltpu.sync_copy(hist_v, accd.at[idx80_v], add=True)

    plsc.subcore_barrier()

    pltpu.sync_copy(accf.at[pl.ds(base, ROWS_PER_SUB)],
                    out_feat.at[pl.ds(base, ROWS_PER_SUB), pl.ds(c * DH, DH)])

    @pl.when(s == 0)
    def _write_deg():
        pltpu.sync_copy(accd, out_deg.at[c])


_sc_segsum = pl.kernel(
    _sc_body,
    out_type=(
        jax.ShapeDtypeStruct((NACC, D), _f32),
        jax.ShapeDtypeStruct((2, HROWS, 128), _f32),
    ),
    mesh=_sc_mesh,
    scratch_types=[
        pltpu.VMEM((SUPER, CHUNK), _i32),
        pltpu.VMEM((SUPER, CHUNK), _i32),
        pltpu.VMEM((CHUNK, DH), _f32),
        pltpu.VMEM((CHUNK, DH), _f32),
        pltpu.VMEM((HROWS, 128), _f32),
        pltpu.VMEM((HROWS,), _i32),
        pltpu.VMEM_SHARED((NTBL, DH), _f32),
        pltpu.VMEM_SHARED((NACC, DH), _f32),
        pltpu.VMEM_SHARED((HROWS, 128), _f32),
        pltpu.SemaphoreType.DMA,
        pltpu.SemaphoreType.DMA,
        pltpu.SemaphoreType.DMA,
        pltpu.SemaphoreType.DMA,
    ],
    compiler_params=pltpu.CompilerParams(use_tc_tiling_on_sc=False,
                                         needs_layout_passes=False),
)


_RB = 1000
_HI = jax.lax.Precision.DEFAULT


def _tc_rmm_body(x_ref, w_ref, o_ref):
    o_ref[...] = jnp.dot(x_ref[...], w_ref[...], preferred_element_type=_f32,
                         precision=_HI)


def _tc_rmm(x, wT):
    return pl.pallas_call(
        _tc_rmm_body,
        grid=(N_NODES // _RB,),
        in_specs=[
            pl.BlockSpec((_RB, D), lambda i: (i, 0)),
            pl.BlockSpec((D, D), lambda i: (0, 0)),
        ],
        out_specs=pl.BlockSpec((_RB, D), lambda i: (i, 0)),
        out_shape=jax.ShapeDtypeStruct((N_NODES, D), _f32),
    )(x, wT)


def _tc_mid_body(pf_ref, dv_ref, b_ref, r_ref, wlT_ref,
                 h_ref, y2_ref):
    inv = 1.0 / jnp.maximum(dv_ref[...], 1.0)
    h = jnp.maximum(pf_ref[...] * inv + b_ref[...] + r_ref[...], 0.0)
    h_ref[...] = h
    y2_ref[...] = jnp.dot(h, wlT_ref[...], preferred_element_type=_f32,
                          precision=_HI)


def _tc_mid(pf, dv, b, r, wlT):
    return pl.pallas_call(
        _tc_mid_body,
        grid=(N_NODES // _RB,),
        in_specs=[
            pl.BlockSpec((_RB, D), lambda i: (i, 0)),
            pl.BlockSpec((_RB, 1), lambda i: (i, 0)),
            pl.BlockSpec((1, D), lambda i: (0, 0)),
            pl.BlockSpec((_RB, D), lambda i: (i, 0)),
            pl.BlockSpec((D, D), lambda i: (0, 0)),
        ],
        out_specs=[
            pl.BlockSpec((_RB, D), lambda i: (i, 0)),
            pl.BlockSpec((_RB, D), lambda i: (i, 0)),
        ],
        out_shape=[
            jax.ShapeDtypeStruct((N_NODES, D), _f32),
            jax.ShapeDtypeStruct((N_NODES, D), _f32),
        ],
    )(pf, dv, b, r, wlT)


def _tc_post_body(pf_ref, dv_ref, b_ref, r_ref, out_ref):
    inv = 1.0 / jnp.maximum(dv_ref[...], 1.0)
    out_ref[0] = pf_ref[...] * inv + b_ref[...] + r_ref[...]


def _tc_post(pf, dv, b, r):
    return pl.pallas_call(
        _tc_post_body,
        grid=(N_NODES // _RB,),
        in_specs=[
            pl.BlockSpec((_RB, D), lambda i: (i, 0)),
            pl.BlockSpec((_RB, 1), lambda i: (i, 0)),
            pl.BlockSpec((1, D), lambda i: (0, 0)),
            pl.BlockSpec((_RB, D), lambda i: (i, 0)),
        ],
        out_specs=pl.BlockSpec((1, _RB, D), lambda i: (i // 5, i % 5, 0)),
        out_shape=jax.ShapeDtypeStruct((2, N_USERS, D), _f32),
    )(pf, dv, b, r)


def kernel(edge_index, num_users, user_emb, prod_emb, W1l, b1l, W1r, W2l,
           b2l, W2r):
    x = jnp.concatenate([user_emb, prod_emb], axis=0)

    ep = jnp.pad(edge_index, ((0, 0), (0, EPAD - EDGES)),
                 constant_values=N_NODES)
    srcp = ep[0].reshape(NSUB, CHUNKS_PER_SUB, CHUNK)
    dstp = ep[1].reshape(NSUB, CHUNKS_PER_SUB, CHUNK)

    zeros80 = jnp.zeros((HROWS, 128), _f32)
    idx80 = jnp.arange(HROWS, dtype=_i32)

    b1 = b1l.reshape(1, D)
    b2 = b2l.reshape(1, D)

    y1 = _tc_rmm(x, W1l.T)
    pf1, pd1 = _sc_segsum(y1, srcp, dstp, zeros80, idx80)
    r1 = _tc_rmm(x, W1r.T)
    degv = pd1[0].reshape(NACC, 1)[:N_NODES]
    h, y2 = _tc_mid(pf1, degv, b1, r1, W2l.T)

    pf2, _ = _sc_segsum(y2, srcp, dstp, zeros80, idx80)
    r2 = _tc_rmm(h, W2r.T)
    out = _tc_post(pf2, degv, b2, r2)

    return (out[0], out[1])
